# baseline (speedup 1.0000x reference)
import jax
import jax.numpy as jnp
from jax.experimental import pallas as pl
from jax.experimental.pallas import tpu as pltpu

B = 2
SEQ = 4096
DIM = 1024
HEADS = 16
HEAD_DIM = DIM // HEADS
WSZ = 256
NC = SEQ // WSZ
SELF_VAL = -50000.0
SCALE = HEAD_DIM ** -0.5
PREC = jax.lax.Precision.HIGHEST

TTILE = 512
ROWS_P2 = 128
CHUNK = 512
CPC = 16


def _proj_kernel(x_ref, wqk_ref, wv_ref, m2_ref, s_ref, rv_ref,
                 qk_ref, v_ref, dists_ref):
    x = x_ref[0]
    wqk = wqk_ref[...]
    wv = wv_ref[...]
    dn = (((1,), (1,)), ((), ()))
    qkt = jax.lax.dot_general(wqk, x, dn, preferred_element_type=jnp.float32)
    vt = jax.lax.dot_general(wv, x, dn, preferred_element_type=jnp.float32)
    qk_ref[0] = qkt.astype(jnp.bfloat16)
    v_ref[0] = vt.astype(jnp.bfloat16)
    dn_cc = (((0,), (0,)), ((), ()))
    sq = jax.lax.dot_general(s_ref[...], qkt * qkt, dn_cc,
                             preferred_element_type=jnp.float32,
                             precision=PREC)
    norm = jnp.sqrt(sq)
    normb = jax.lax.dot_general(rv_ref[...], norm, dn_cc,
                                preferred_element_type=jnp.float32,
                                precision=PREC)
    kn = qkt / jnp.maximum(normb, 1e-12)
    dists_ref[0] = jax.lax.dot_general(m2_ref[...], kn, dn_cc,
                                       preferred_element_type=jnp.float32)


def _cumsum_rows(x, ltri):
    outs = []
    off = jnp.zeros((x.shape[0], 1), jnp.float32)
    for j in range(SEQ // CHUNK):
        c = x[:, j * CHUNK:(j + 1) * CHUNK]
        cs = jnp.dot(c, ltri, preferred_element_type=jnp.float32) + off
        off = cs[:, CHUNK - 1:CHUNK]
        outs.append(cs)
    return jnp.concatenate(outs, axis=1)


def _topk_kernel(dt_ref, ltri_ref, agg_ref, rank_ref, denom_ref):
    x = dt_ref[...]
    ltri = ltri_ref[...]
    ii = jax.lax.bitcast_convert_type(x, jnp.int32)
    s = jnp.where(ii < 0, ii ^ jnp.int32(0x7FFFFFFF), ii)
    kf = jnp.float32(WSZ)
    cnt_nn = jnp.sum(jnp.where(s >= 0, 1.0, 0.0), axis=1, keepdims=True)
    t = jnp.where(cnt_nn >= kf, jnp.int32(0), jnp.int32(-2147483648))
    for bit in range(30, -1, -1):
        cand = t + jnp.int32(1 << bit)
        cnt = jnp.sum(jnp.where(s >= cand, 1.0, 0.0), axis=1, keepdims=True)
        t = jnp.where(cnt >= kf, cand, t)
    gt = jnp.where(s > t, 1.0, 0.0)
    eq = jnp.where(s == t, 1.0, 0.0)
    need = kf - jnp.sum(gt, axis=1, keepdims=True)
    cumeq = _cumsum_rows(eq, ltri)
    sel = gt + eq * jnp.where(cumeq <= need, 1.0, 0.0)
    ranks = _cumsum_rows(sel, ltri) * sel
    rank_ref[...] = ranks
    denom_ref[...] = jnp.dot(agg_ref[...], sel,
                             preferred_element_type=jnp.float32)


def _attn_kernel(rank_ref, qkt_ref, vt_ref, rw_ref, numer_ref):
    cg = pl.program_id(2)
    bf16 = jnp.bfloat16
    R = CPC * WSZ
    qkt = qkt_ref[0, 0]
    vtt = vt_ref[0, 0]
    rw = rw_ref[0]
    dn_rr = (((1,), (1,)), ((), ()))
    dn_cc = (((0,), (0,)), ((), ()))
    dn_cr = (((0,), (1,)), ((), ()))
    rr = rank_ref[...].reshape(CPC, 1, SEQ).astype(bf16)
    r_rep = jnp.broadcast_to(rr, (CPC, WSZ, SEQ)).reshape(R, SEQ)
    p1 = ((jax.lax.broadcasted_iota(jnp.int32, (R, SEQ), 0) & (WSZ - 1)) + 1
          ).astype(bf16)
    g = jnp.where(p1 == r_rep, bf16(1.0), bf16(0.0))
    qt = jax.lax.dot_general(qkt, g, dn_rr,
                             preferred_element_type=jnp.float32)
    vt = jax.lax.dot_general(vtt, g, dn_rr,
                             preferred_element_type=jnp.float32)
    kn = jnp.sqrt(jnp.sum(qt * qt, axis=0, keepdims=True))
    kt = qt / jnp.maximum(kn, 1e-12)
    dots_w = []
    xpad_w = []
    zeros = jnp.zeros((WSZ, WSZ), jnp.float32)
    for cc in range(CPC):
        sl = slice(cc * WSZ, (cc + 1) * WSZ)
        qtc = qt[:, sl]
        dots_w.append(jax.lax.dot_general(
            qtc, kt[:, sl], dn_cc, preferred_element_type=jnp.float32))
        qr = jax.lax.dot_general(qtc, rw, dn_cr,
                                 preferred_element_type=jnp.float32)
        xpad_w.append(jnp.concatenate([qr, zeros], axis=1) if cc % 2 == 0
                      else jnp.concatenate([zeros, qr], axis=1))
    dots = jnp.concatenate(dots_w, axis=0) * SCALE
    xpad = jnp.concatenate(xpad_w, axis=0) * SCALE
    rel = pltpu.roll(xpad, shift=WSZ + 1, axis=1, stride=1,
                     stride_axis=0)[:, :WSZ]
    dots = dots + rel
    ri = jax.lax.broadcasted_iota(jnp.int32, (R, WSZ), 0) & (WSZ - 1)
    ci = jax.lax.broadcasted_iota(jnp.int32, (R, WSZ), 1)
    dots = jnp.where(ri == ci, SELF_VAL, dots)
    m = jnp.max(dots, axis=1, keepdims=True)
    e = jnp.exp(dots - m)
    p = (e / jnp.sum(e, axis=1, keepdims=True)).astype(bf16)
    vtb = vt.astype(bf16)
    bot_w = [jax.lax.dot_general(vtb[:, cc * WSZ:(cc + 1) * WSZ],
                                 p[cc * WSZ:(cc + 1) * WSZ], dn_rr,
                                 preferred_element_type=jnp.float32)
             for cc in range(CPC)]
    bot = jnp.concatenate(bot_w, axis=1).astype(bf16)
    acc = jnp.dot(bot, g, preferred_element_type=jnp.float32)

    @pl.when(cg == 0)
    def _():
        numer_ref[0, 0] = acc

    @pl.when(cg != 0)
    def _():
        numer_ref[0, 0] += acc


def _out_kernel(numer_ref, denom_ref, rv_ref, wo_ref, out_ref):
    d = denom_ref[...]
    dn_cc = (((0,), (0,)), ((), ()))
    denombt = jax.lax.dot_general(rv_ref[...], d, dn_cc,
                                  preferred_element_type=jnp.float32,
                                  precision=PREC)
    yt = numer_ref[0] / (denombt + 1e-5)
    dn_out = (((0,), (1,)), ((), ()))
    out_ref[0] = jax.lax.dot_general(yt, wo_ref[...], dn_out,
                                     preferred_element_type=jnp.float32)


@jax.jit
def kernel(x, Wqk, Wv, Wo, rel_w, means):
    f32 = jnp.float32
    eye = jnp.eye(HEADS, dtype=f32)
    m2 = jnp.einsum('hcd,hk->hdkc', means, eye).reshape(DIM, HEADS * NC)
    s_mat = jnp.repeat(eye, HEAD_DIM, axis=0)
    r16 = jnp.repeat(eye, NC, axis=1)
    rv = jnp.repeat(eye, HEAD_DIM, axis=1)
    ltri = jnp.tril(jnp.ones((CHUNK, CHUNK), f32)).T
    agg = jnp.repeat(jnp.eye(ROWS_P2 // NC, dtype=f32), NC, axis=1)
    rwt = jnp.transpose(rel_w, (1, 0, 2))

    nt = SEQ // TTILE
    qk, v, dists = pl.pallas_call(
        _proj_kernel,
        grid=(B, nt),
        in_specs=[
            pl.BlockSpec((1, TTILE, DIM), lambda b, t: (b, t, 0)),
            pl.BlockSpec((DIM, DIM), lambda b, t: (0, 0)),
            pl.BlockSpec((DIM, DIM), lambda b, t: (0, 0)),
            pl.BlockSpec((DIM, HEADS * NC), lambda b, t: (0, 0)),
            pl.BlockSpec((DIM, HEADS), lambda b, t: (0, 0)),
            pl.BlockSpec((HEADS, DIM), lambda b, t: (0, 0)),
        ],
        out_specs=[
            pl.BlockSpec((1, DIM, TTILE), lambda b, t: (b, 0, t)),
            pl.BlockSpec((1, DIM, TTILE), lambda b, t: (b, 0, t)),
            pl.BlockSpec((1, HEADS * NC, TTILE), lambda b, t: (b, 0, t)),
        ],
        out_shape=[
            jax.ShapeDtypeStruct((B, DIM, SEQ), jnp.bfloat16),
            jax.ShapeDtypeStruct((B, DIM, SEQ), jnp.bfloat16),
            jax.ShapeDtypeStruct((B, HEADS * NC, SEQ), f32),
        ],
        compiler_params=pltpu.CompilerParams(
            dimension_semantics=("parallel", "parallel")),
    )(x, Wqk, Wv, m2, s_mat, rv)

    dt = dists.reshape(B * HEADS * NC, SEQ)

    nrows = B * HEADS * NC
    ranks, denom = pl.pallas_call(
        _topk_kernel,
        grid=(nrows // ROWS_P2,),
        in_specs=[
            pl.BlockSpec((ROWS_P2, SEQ), lambda i: (i, 0)),
            pl.BlockSpec((CHUNK, CHUNK), lambda i: (0, 0)),
            pl.BlockSpec((ROWS_P2 // NC, ROWS_P2), lambda i: (0, 0)),
        ],
        out_specs=[
            pl.BlockSpec((ROWS_P2, SEQ), lambda i: (i, 0)),
            pl.BlockSpec((ROWS_P2 // NC, SEQ), lambda i: (i, 0)),
        ],
        out_shape=[
            jax.ShapeDtypeStruct((nrows, SEQ), f32),
            jax.ShapeDtypeStruct((B * HEADS, SEQ), f32),
        ],
        compiler_params=pltpu.CompilerParams(
            dimension_semantics=("parallel",)),
    )(dt, ltri, agg)

    ranks3 = ranks.reshape(nrows, 1, SEQ)
    qk4 = qk.reshape(B, HEADS, HEAD_DIM, SEQ)
    v4 = v.reshape(B, HEADS, HEAD_DIM, SEQ)
    numert = pl.pallas_call(
        _attn_kernel,
        grid=(B, HEADS, NC // CPC),
        in_specs=[
            pl.BlockSpec((CPC, 1, SEQ),
                         lambda b, h, c: (b * (HEADS * NC // CPC)
                                          + h * (NC // CPC) + c, 0, 0)),
            pl.BlockSpec((1, 1, HEAD_DIM, SEQ), lambda b, h, c: (b, h, 0, 0)),
            pl.BlockSpec((1, 1, HEAD_DIM, SEQ), lambda b, h, c: (b, h, 0, 0)),
            pl.BlockSpec((1, WSZ, HEAD_DIM), lambda b, h, c: (h, 0, 0)),
        ],
        out_specs=pl.BlockSpec((1, 1, HEAD_DIM, SEQ),
                               lambda b, h, c: (b, h, 0, 0)),
        out_shape=jax.ShapeDtypeStruct((B, HEADS, HEAD_DIM, SEQ), f32),
        compiler_params=pltpu.CompilerParams(
            dimension_semantics=("parallel", "parallel", "arbitrary")),
    )(ranks3, qk4, v4, rwt)
    numer3 = numert.reshape(B, DIM, SEQ)

    out = pl.pallas_call(
        _out_kernel,
        grid=(B, nt),
        in_specs=[
            pl.BlockSpec((1, DIM, TTILE), lambda b, t: (b, 0, t)),
            pl.BlockSpec((HEADS, TTILE), lambda b, t: (b, t)),
            pl.BlockSpec((HEADS, DIM), lambda b, t: (0, 0)),
            pl.BlockSpec((DIM, DIM), lambda b, t: (0, 0)),
        ],
        out_specs=pl.BlockSpec((1, TTILE, DIM), lambda b, t: (b, t, 0)),
        out_shape=jax.ShapeDtypeStruct((B, SEQ, DIM), f32),
        compiler_params=pltpu.CompilerParams(
            dimension_semantics=("parallel", "parallel")),
    )(numer3, denom.reshape(B * HEADS, SEQ), rv, Wo)
    return out


# speedup vs baseline: 1.0130x; 1.0130x over previous
import jax
import jax.numpy as jnp
from jax.experimental import pallas as pl
from jax.experimental.pallas import tpu as pltpu

B = 2
SEQ = 4096
DIM = 1024
HEADS = 16
HEAD_DIM = DIM // HEADS
WSZ = 256
NC = SEQ // WSZ
SELF_VAL = -50000.0
SCALE = HEAD_DIM ** -0.5
PREC = jax.lax.Precision.HIGHEST

TTILE = 512
ROWS_P2 = 128
CHUNK = 512
CPC = 8


def _proj_kernel(x_ref, wqk_ref, wv_ref, m2_ref, s_ref, rv_ref,
                 qk_ref, v_ref, dists_ref):
    x = x_ref[0]
    wqk = wqk_ref[...]
    wv = wv_ref[...]
    dn = (((1,), (1,)), ((), ()))
    qkt = jax.lax.dot_general(wqk, x, dn, preferred_element_type=jnp.float32)
    vt = jax.lax.dot_general(wv, x, dn, preferred_element_type=jnp.float32)
    qk_ref[0] = qkt.astype(jnp.bfloat16)
    v_ref[0] = vt.astype(jnp.bfloat16)
    dn_cc = (((0,), (0,)), ((), ()))
    sq = jax.lax.dot_general(s_ref[...], qkt * qkt, dn_cc,
                             preferred_element_type=jnp.float32,
                             precision=PREC)
    norm = jnp.sqrt(sq)
    normb = jax.lax.dot_general(rv_ref[...], norm, dn_cc,
                                preferred_element_type=jnp.float32,
                                precision=PREC)
    kn = qkt / jnp.maximum(normb, 1e-12)
    dists_ref[0] = jax.lax.dot_general(m2_ref[...], kn, dn_cc,
                                       preferred_element_type=jnp.float32)


def _cumsum_rows(x, ltri):
    outs = []
    off = jnp.zeros((x.shape[0], 1), jnp.float32)
    for j in range(SEQ // CHUNK):
        c = x[:, j * CHUNK:(j + 1) * CHUNK]
        cs = jnp.dot(c, ltri, preferred_element_type=jnp.float32) + off
        off = cs[:, CHUNK - 1:CHUNK]
        outs.append(cs)
    return jnp.concatenate(outs, axis=1)


def _topk_kernel(dt_ref, ltri_ref, agg_ref, rank_ref, denom_ref):
    x = dt_ref[...]
    ltri = ltri_ref[...]
    ii = jax.lax.bitcast_convert_type(x, jnp.int32)
    s = jnp.where(ii < 0, ii ^ jnp.int32(0x7FFFFFFF), ii)
    kf = jnp.float32(WSZ)
    cnt_nn = jnp.sum(jnp.where(s >= 0, 1.0, 0.0), axis=1, keepdims=True)
    t = jnp.where(cnt_nn >= kf, jnp.int32(0), jnp.int32(-2147483648))
    for bit in range(30, -1, -1):
        cand = t + jnp.int32(1 << bit)
        cnt = jnp.sum(jnp.where(s >= cand, 1.0, 0.0), axis=1, keepdims=True)
        t = jnp.where(cnt >= kf, cand, t)
    gt = jnp.where(s > t, 1.0, 0.0)
    eq = jnp.where(s == t, 1.0, 0.0)
    need = kf - jnp.sum(gt, axis=1, keepdims=True)
    cumeq = _cumsum_rows(eq, ltri)
    sel = gt + eq * jnp.where(cumeq <= need, 1.0, 0.0)
    ranks = _cumsum_rows(sel, ltri) * sel
    rank_ref[...] = ranks
    denom_ref[...] = jnp.dot(agg_ref[...], sel,
                             preferred_element_type=jnp.float32)


def _attn_kernel(rank_ref, qkt_ref, vt_ref, rw_ref, numer_ref):
    cg = pl.program_id(2)
    bf16 = jnp.bfloat16
    R = CPC * WSZ
    qkt = qkt_ref[0, 0]
    vtt = vt_ref[0, 0]
    rw = rw_ref[0]
    dn_rr = (((1,), (1,)), ((), ()))
    dn_cc = (((0,), (0,)), ((), ()))
    dn_cr = (((0,), (1,)), ((), ()))
    rr = rank_ref[...].reshape(CPC, 1, SEQ).astype(bf16)
    r_rep = jnp.broadcast_to(rr, (CPC, WSZ, SEQ)).reshape(R, SEQ)
    p1 = ((jax.lax.broadcasted_iota(jnp.int32, (R, SEQ), 0) & (WSZ - 1)) + 1
          ).astype(bf16)
    g = jnp.where(p1 == r_rep, bf16(1.0), bf16(0.0))
    qt = jax.lax.dot_general(qkt, g, dn_rr,
                             preferred_element_type=jnp.float32)
    vt = jax.lax.dot_general(vtt, g, dn_rr,
                             preferred_element_type=jnp.float32)
    kn = jnp.sqrt(jnp.sum(qt * qt, axis=0, keepdims=True))
    kt = qt / jnp.maximum(kn, 1e-12)
    dots_w = []
    xpad_w = []
    zeros = jnp.zeros((WSZ, WSZ), jnp.float32)
    for cc in range(CPC):
        sl = slice(cc * WSZ, (cc + 1) * WSZ)
        qtc = qt[:, sl]
        dots_w.append(jax.lax.dot_general(
            qtc, kt[:, sl], dn_cc, preferred_element_type=jnp.float32))
        qr = jax.lax.dot_general(qtc, rw, dn_cr,
                                 preferred_element_type=jnp.float32)
        xpad_w.append(jnp.concatenate([qr, zeros], axis=1) if cc % 2 == 0
                      else jnp.concatenate([zeros, qr], axis=1))
    dots = jnp.concatenate(dots_w, axis=0) * SCALE
    xpad = jnp.concatenate(xpad_w, axis=0) * SCALE
    rel = pltpu.roll(xpad, shift=WSZ + 1, axis=1, stride=1,
                     stride_axis=0)[:, :WSZ]
    dots = dots + rel
    ri = jax.lax.broadcasted_iota(jnp.int32, (R, WSZ), 0) & (WSZ - 1)
    ci = jax.lax.broadcasted_iota(jnp.int32, (R, WSZ), 1)
    dots = jnp.where(ri == ci, SELF_VAL, dots)
    m = jnp.max(dots, axis=1, keepdims=True)
    e = jnp.exp(dots - m)
    p = (e / jnp.sum(e, axis=1, keepdims=True)).astype(bf16)
    vtb = vt.astype(bf16)
    bot_w = [jax.lax.dot_general(vtb[:, cc * WSZ:(cc + 1) * WSZ],
                                 p[cc * WSZ:(cc + 1) * WSZ], dn_rr,
                                 preferred_element_type=jnp.float32)
             for cc in range(CPC)]
    bot = jnp.concatenate(bot_w, axis=1).astype(bf16)
    acc = jnp.dot(bot, g, preferred_element_type=jnp.float32)

    @pl.when(cg == 0)
    def _():
        numer_ref[0, 0] = acc

    @pl.when(cg != 0)
    def _():
        numer_ref[0, 0] += acc


def _out_kernel(numer_ref, denom_ref, rv_ref, wo_ref, out_ref):
    d = denom_ref[...]
    dn_cc = (((0,), (0,)), ((), ()))
    denombt = jax.lax.dot_general(rv_ref[...], d, dn_cc,
                                  preferred_element_type=jnp.float32,
                                  precision=PREC)
    yt = numer_ref[0] / (denombt + 1e-5)
    dn_out = (((0,), (1,)), ((), ()))
    out_ref[0] = jax.lax.dot_general(yt, wo_ref[...], dn_out,
                                     preferred_element_type=jnp.float32)


@jax.jit
def kernel(x, Wqk, Wv, Wo, rel_w, means):
    f32 = jnp.float32
    eye = jnp.eye(HEADS, dtype=f32)
    m2 = jnp.einsum('hcd,hk->hdkc', means, eye).reshape(DIM, HEADS * NC)
    s_mat = jnp.repeat(eye, HEAD_DIM, axis=0)
    r16 = jnp.repeat(eye, NC, axis=1)
    rv = jnp.repeat(eye, HEAD_DIM, axis=1)
    ltri = jnp.tril(jnp.ones((CHUNK, CHUNK), f32)).T
    agg = jnp.repeat(jnp.eye(ROWS_P2 // NC, dtype=f32), NC, axis=1)
    rwt = jnp.transpose(rel_w, (1, 0, 2))

    nt = SEQ // TTILE
    qk, v, dists = pl.pallas_call(
        _proj_kernel,
        grid=(B, nt),
        in_specs=[
            pl.BlockSpec((1, TTILE, DIM), lambda b, t: (b, t, 0)),
            pl.BlockSpec((DIM, DIM), lambda b, t: (0, 0)),
            pl.BlockSpec((DIM, DIM), lambda b, t: (0, 0)),
            pl.BlockSpec((DIM, HEADS * NC), lambda b, t: (0, 0)),
            pl.BlockSpec((DIM, HEADS), lambda b, t: (0, 0)),
            pl.BlockSpec((HEADS, DIM), lambda b, t: (0, 0)),
        ],
        out_specs=[
            pl.BlockSpec((1, DIM, TTILE), lambda b, t: (b, 0, t)),
            pl.BlockSpec((1, DIM, TTILE), lambda b, t: (b, 0, t)),
            pl.BlockSpec((1, HEADS * NC, TTILE), lambda b, t: (b, 0, t)),
        ],
        out_shape=[
            jax.ShapeDtypeStruct((B, DIM, SEQ), jnp.bfloat16),
            jax.ShapeDtypeStruct((B, DIM, SEQ), jnp.bfloat16),
            jax.ShapeDtypeStruct((B, HEADS * NC, SEQ), f32),
        ],
        compiler_params=pltpu.CompilerParams(
            dimension_semantics=("parallel", "parallel")),
    )(x, Wqk, Wv, m2, s_mat, rv)

    dt = dists.reshape(B * HEADS * NC, SEQ)

    nrows = B * HEADS * NC
    ranks, denom = pl.pallas_call(
        _topk_kernel,
        grid=(nrows // ROWS_P2,),
        in_specs=[
            pl.BlockSpec((ROWS_P2, SEQ), lambda i: (i, 0)),
            pl.BlockSpec((CHUNK, CHUNK), lambda i: (0, 0)),
            pl.BlockSpec((ROWS_P2 // NC, ROWS_P2), lambda i: (0, 0)),
        ],
        out_specs=[
            pl.BlockSpec((ROWS_P2, SEQ), lambda i: (i, 0)),
            pl.BlockSpec((ROWS_P2 // NC, SEQ), lambda i: (i, 0)),
        ],
        out_shape=[
            jax.ShapeDtypeStruct((nrows, SEQ), f32),
            jax.ShapeDtypeStruct((B * HEADS, SEQ), f32),
        ],
        compiler_params=pltpu.CompilerParams(
            dimension_semantics=("parallel",)),
    )(dt, ltri, agg)

    ranks3 = ranks.reshape(nrows, 1, SEQ)
    qk4 = qk.reshape(B, HEADS, HEAD_DIM, SEQ)
    v4 = v.reshape(B, HEADS, HEAD_DIM, SEQ)
    numert = pl.pallas_call(
        _attn_kernel,
        grid=(B, HEADS, NC // CPC),
        in_specs=[
            pl.BlockSpec((CPC, 1, SEQ),
                         lambda b, h, c: (b * (HEADS * NC // CPC)
                                          + h * (NC // CPC) + c, 0, 0)),
            pl.BlockSpec((1, 1, HEAD_DIM, SEQ), lambda b, h, c: (b, h, 0, 0)),
            pl.BlockSpec((1, 1, HEAD_DIM, SEQ), lambda b, h, c: (b, h, 0, 0)),
            pl.BlockSpec((1, WSZ, HEAD_DIM), lambda b, h, c: (h, 0, 0)),
        ],
        out_specs=pl.BlockSpec((1, 1, HEAD_DIM, SEQ),
                               lambda b, h, c: (b, h, 0, 0)),
        out_shape=jax.ShapeDtypeStruct((B, HEADS, HEAD_DIM, SEQ), f32),
        compiler_params=pltpu.CompilerParams(
            dimension_semantics=("parallel", "parallel", "arbitrary")),
    )(ranks3, qk4, v4, rwt)
    numer3 = numert.reshape(B, DIM, SEQ)

    out = pl.pallas_call(
        _out_kernel,
        grid=(B, nt),
        in_specs=[
            pl.BlockSpec((1, DIM, TTILE), lambda b, t: (b, 0, t)),
            pl.BlockSpec((HEADS, TTILE), lambda b, t: (b, t)),
            pl.BlockSpec((HEADS, DIM), lambda b, t: (0, 0)),
            pl.BlockSpec((DIM, DIM), lambda b, t: (0, 0)),
        ],
        out_specs=pl.BlockSpec((1, TTILE, DIM), lambda b, t: (b, t, 0)),
        out_shape=jax.ShapeDtypeStruct((B, SEQ, DIM), f32),
        compiler_params=pltpu.CompilerParams(
            dimension_semantics=("parallel", "parallel")),
    )(numer3, denom.reshape(B * HEADS, SEQ), rv, Wo)
    return out


# speedup vs baseline: 1.0135x; 1.0005x over previous
import jax
import jax.numpy as jnp
from jax.experimental import pallas as pl
from jax.experimental.pallas import tpu as pltpu

B = 2
SEQ = 4096
DIM = 1024
HEADS = 16
HEAD_DIM = DIM // HEADS
WSZ = 256
NC = SEQ // WSZ
SELF_VAL = -50000.0
SCALE = HEAD_DIM ** -0.5
PREC = jax.lax.Precision.HIGHEST

TTILE = 512
ROWS_P2 = 128
CHUNK = 512
CPC = 8


def _proj_kernel(x_ref, wqk_ref, wv_ref, m2_ref, s_ref, rv_ref,
                 qk_ref, v_ref, dists_ref):
    x = x_ref[0]
    wqk = wqk_ref[...]
    wv = wv_ref[...]
    dn = (((1,), (1,)), ((), ()))
    qkt = jax.lax.dot_general(wqk, x, dn, preferred_element_type=jnp.float32)
    vt = jax.lax.dot_general(wv, x, dn, preferred_element_type=jnp.float32)
    qk_ref[0] = qkt.astype(jnp.bfloat16)
    v_ref[0] = vt.astype(jnp.bfloat16)
    dn_cc = (((0,), (0,)), ((), ()))
    sq = jax.lax.dot_general(s_ref[...], qkt * qkt, dn_cc,
                             preferred_element_type=jnp.float32,
                             precision=PREC)
    norm = jnp.sqrt(sq)
    normb = jax.lax.dot_general(rv_ref[...], norm, dn_cc,
                                preferred_element_type=jnp.float32,
                                precision=PREC)
    kn = qkt / jnp.maximum(normb, 1e-12)
    dists_ref[0] = jax.lax.dot_general(m2_ref[...], kn, dn_cc,
                                       preferred_element_type=jnp.float32)


def _cumsum_rows(x, ltri):
    outs = []
    off = jnp.zeros((x.shape[0], 1), jnp.float32)
    for j in range(SEQ // CHUNK):
        c = x[:, j * CHUNK:(j + 1) * CHUNK]
        cs = jnp.dot(c, ltri, preferred_element_type=jnp.float32) + off
        off = cs[:, CHUNK - 1:CHUNK]
        outs.append(cs)
    return jnp.concatenate(outs, axis=1)


def _topk_kernel(dt_ref, ltri_ref, agg_ref, rank_ref, denom_ref):
    x = dt_ref[...]
    ltri = ltri_ref[...]
    ii = jax.lax.bitcast_convert_type(x, jnp.int32)
    s = jnp.where(ii < 0, ii ^ jnp.int32(0x7FFFFFFF), ii)
    kf = jnp.float32(WSZ)
    cnt_nn = jnp.sum(jnp.where(s >= 0, 1.0, 0.0), axis=1, keepdims=True)
    t = jnp.where(cnt_nn >= kf, jnp.int32(0), jnp.int32(-2147483648))
    for bit in range(30, -1, -1):
        cand = t + jnp.int32(1 << bit)
        cnt = jnp.sum(jnp.where(s >= cand, 1.0, 0.0), axis=1, keepdims=True)
        t = jnp.where(cnt >= kf, cand, t)
    gt = jnp.where(s > t, 1.0, 0.0)
    eq = jnp.where(s == t, 1.0, 0.0)
    need = kf - jnp.sum(gt, axis=1, keepdims=True)
    cumeq = _cumsum_rows(eq, ltri)
    sel = gt + eq * jnp.where(cumeq <= need, 1.0, 0.0)
    ranks = _cumsum_rows(sel, ltri) * sel
    rank_ref[...] = ranks
    denom_ref[...] = jnp.dot(agg_ref[...], sel,
                             preferred_element_type=jnp.float32)


def _attn_kernel(rank_ref, qkt_ref, vt_ref, rw_ref, numer_ref):
    cg = pl.program_id(2)
    bf16 = jnp.bfloat16
    R = CPC * WSZ
    qkt = qkt_ref[0, 0]
    vtt = vt_ref[0, 0]
    rw = rw_ref[0]
    dn_rr = (((1,), (1,)), ((), ()))
    dn_cc = (((0,), (0,)), ((), ()))
    dn_cr = (((0,), (1,)), ((), ()))
    rr = rank_ref[...].reshape(CPC, 1, SEQ).astype(bf16)
    r_rep = jnp.broadcast_to(rr, (CPC, WSZ, SEQ)).reshape(R, SEQ)
    p1 = ((jax.lax.broadcasted_iota(jnp.int32, (R, SEQ), 0) & (WSZ - 1)) + 1
          ).astype(bf16)
    g = jnp.where(p1 == r_rep, bf16(1.0), bf16(0.0))
    qt = jax.lax.dot_general(qkt, g, dn_rr,
                             preferred_element_type=jnp.float32)
    vt = jax.lax.dot_general(vtt, g, dn_rr,
                             preferred_element_type=jnp.float32)
    kn = jnp.sqrt(jnp.sum(qt * qt, axis=0, keepdims=True))
    kt = qt / jnp.maximum(kn, 1e-12)
    dots_w = []
    xpad_w = []
    zeros = jnp.zeros((WSZ, WSZ), jnp.float32)
    for cc in range(CPC):
        sl = slice(cc * WSZ, (cc + 1) * WSZ)
        qtc = qt[:, sl]
        dots_w.append(jax.lax.dot_general(
            qtc, kt[:, sl], dn_cc, preferred_element_type=jnp.float32))
        qr = jax.lax.dot_general(qtc, rw, dn_cr,
                                 preferred_element_type=jnp.float32)
        xpad_w.append(jnp.concatenate([qr, zeros], axis=1) if cc % 2 == 0
                      else jnp.concatenate([zeros, qr], axis=1))
    dots = jnp.concatenate(dots_w, axis=0) * SCALE
    xpad = jnp.concatenate(xpad_w, axis=0) * SCALE
    rel = pltpu.roll(xpad, shift=WSZ + 1, axis=1, stride=1,
                     stride_axis=0)[:, :WSZ]
    dots = dots + rel
    ri = jax.lax.broadcasted_iota(jnp.int32, (R, WSZ), 0) & (WSZ - 1)
    ci = jax.lax.broadcasted_iota(jnp.int32, (R, WSZ), 1)
    dots = jnp.where(ri == ci, SELF_VAL, dots)
    m = jnp.max(dots, axis=1, keepdims=True)
    e = jnp.exp(dots - m)
    p = (e / jnp.sum(e, axis=1, keepdims=True)).astype(bf16)
    vtb = vt.astype(bf16)
    bot_w = [jax.lax.dot_general(vtb[:, cc * WSZ:(cc + 1) * WSZ],
                                 p[cc * WSZ:(cc + 1) * WSZ], dn_rr,
                                 preferred_element_type=jnp.float32)
             for cc in range(CPC)]
    bot = jnp.concatenate(bot_w, axis=1).astype(bf16)
    acc = jnp.dot(bot, g, preferred_element_type=jnp.float32)

    @pl.when(cg == 0)
    def _():
        numer_ref[0, 0] = acc

    @pl.when(cg != 0)
    def _():
        numer_ref[0, 0] += acc


def _out_kernel(numer_ref, denom_ref, rv_ref, wo_ref, out_ref):
    d = denom_ref[...]
    dn_cc = (((0,), (0,)), ((), ()))
    denombt = jax.lax.dot_general(rv_ref[...], d, dn_cc,
                                  preferred_element_type=jnp.float32,
                                  precision=PREC)
    yt = numer_ref[0] / (denombt + 1e-5)
    dn_out = (((0,), (1,)), ((), ()))
    out_ref[0] = jax.lax.dot_general(yt, wo_ref[...], dn_out,
                                     preferred_element_type=jnp.float32)


@jax.jit
def kernel(x, Wqk, Wv, Wo, rel_w, means):
    f32 = jnp.float32
    eye = jnp.eye(HEADS, dtype=f32)
    m2 = jnp.einsum('hcd,hk->hdkc', means, eye).reshape(DIM, HEADS * NC)
    s_mat = jnp.repeat(eye, HEAD_DIM, axis=0)
    rv = jnp.repeat(eye, HEAD_DIM, axis=1)
    ltri = jnp.tril(jnp.ones((CHUNK, CHUNK), f32)).T
    agg = jnp.repeat(jnp.eye(ROWS_P2 // NC, dtype=f32), NC, axis=1)
    rwt = jnp.transpose(rel_w, (1, 0, 2))

    nt = SEQ // TTILE
    qk, v, dists = pl.pallas_call(
        _proj_kernel,
        grid=(B, nt),
        in_specs=[
            pl.BlockSpec((1, TTILE, DIM), lambda b, t: (b, t, 0)),
            pl.BlockSpec((DIM, DIM), lambda b, t: (0, 0)),
            pl.BlockSpec((DIM, DIM), lambda b, t: (0, 0)),
            pl.BlockSpec((DIM, HEADS * NC), lambda b, t: (0, 0)),
            pl.BlockSpec((DIM, HEADS), lambda b, t: (0, 0)),
            pl.BlockSpec((HEADS, DIM), lambda b, t: (0, 0)),
        ],
        out_specs=[
            pl.BlockSpec((1, DIM, TTILE), lambda b, t: (b, 0, t)),
            pl.BlockSpec((1, DIM, TTILE), lambda b, t: (b, 0, t)),
            pl.BlockSpec((1, HEADS * NC, TTILE), lambda b, t: (b, 0, t)),
        ],
        out_shape=[
            jax.ShapeDtypeStruct((B, DIM, SEQ), jnp.bfloat16),
            jax.ShapeDtypeStruct((B, DIM, SEQ), jnp.bfloat16),
            jax.ShapeDtypeStruct((B, HEADS * NC, SEQ), f32),
        ],
        compiler_params=pltpu.CompilerParams(
            dimension_semantics=("parallel", "parallel")),
    )(x, Wqk, Wv, m2, s_mat, rv)

    dt = dists.reshape(B * HEADS * NC, SEQ)

    nrows = B * HEADS * NC
    ranks, denom = pl.pallas_call(
        _topk_kernel,
        grid=(nrows // ROWS_P2,),
        in_specs=[
            pl.BlockSpec((ROWS_P2, SEQ), lambda i: (i, 0)),
            pl.BlockSpec((CHUNK, CHUNK), lambda i: (0, 0)),
            pl.BlockSpec((ROWS_P2 // NC, ROWS_P2), lambda i: (0, 0)),
        ],
        out_specs=[
            pl.BlockSpec((ROWS_P2, SEQ), lambda i: (i, 0)),
            pl.BlockSpec((ROWS_P2 // NC, SEQ), lambda i: (i, 0)),
        ],
        out_shape=[
            jax.ShapeDtypeStruct((nrows, SEQ), f32),
            jax.ShapeDtypeStruct((B * HEADS, SEQ), f32),
        ],
        compiler_params=pltpu.CompilerParams(
            dimension_semantics=("parallel",)),
    )(dt, ltri, agg)

    ranks3 = ranks.reshape(nrows, 1, SEQ)
    qk4 = qk.reshape(B, HEADS, HEAD_DIM, SEQ)
    v4 = v.reshape(B, HEADS, HEAD_DIM, SEQ)
    numert = pl.pallas_call(
        _attn_kernel,
        grid=(B, HEADS, NC // CPC),
        in_specs=[
            pl.BlockSpec((CPC, 1, SEQ),
                         lambda b, h, c: (b * (HEADS * NC // CPC)
                                          + h * (NC // CPC) + c, 0, 0)),
            pl.BlockSpec((1, 1, HEAD_DIM, SEQ), lambda b, h, c: (b, h, 0, 0)),
            pl.BlockSpec((1, 1, HEAD_DIM, SEQ), lambda b, h, c: (b, h, 0, 0)),
            pl.BlockSpec((1, WSZ, HEAD_DIM), lambda b, h, c: (h, 0, 0)),
        ],
        out_specs=pl.BlockSpec((1, 1, HEAD_DIM, SEQ),
                               lambda b, h, c: (b, h, 0, 0)),
        out_shape=jax.ShapeDtypeStruct((B, HEADS, HEAD_DIM, SEQ), f32),
        compiler_params=pltpu.CompilerParams(
            dimension_semantics=("parallel", "parallel", "arbitrary")),
    )(ranks3, qk4, v4, rwt)
    numer3 = numert.reshape(B, DIM, SEQ)

    out = pl.pallas_call(
        _out_kernel,
        grid=(B, nt),
        in_specs=[
            pl.BlockSpec((1, DIM, TTILE), lambda b, t: (b, 0, t)),
            pl.BlockSpec((HEADS, TTILE), lambda b, t: (b, t)),
            pl.BlockSpec((HEADS, DIM), lambda b, t: (0, 0)),
            pl.BlockSpec((DIM, DIM), lambda b, t: (0, 0)),
        ],
        out_specs=pl.BlockSpec((1, TTILE, DIM), lambda b, t: (b, t, 0)),
        out_shape=jax.ShapeDtypeStruct((B, SEQ, DIM), f32),
        compiler_params=pltpu.CompilerParams(
            dimension_semantics=("parallel", "parallel")),
    )(numer3, denom.reshape(B * HEADS, SEQ), rv, Wo)
    return out
